# Initial kernel scaffold; baseline (speedup 1.0000x reference)
import jax
import jax.numpy as jnp
from jax.experimental import pallas as pl


def kernel(x, feature, A, Bmap, kernels, one_padding, mlp_w, mlp_b, conv1_w, conv1_b, bn_gamma, bn_beta, mlp_out_w):
    raise NotImplementedError("write your pallas kernel here")


# speedup vs baseline: 1.8632x; 1.8632x over previous
import functools
import math

import jax
import jax.numpy as jnp
import numpy as np
from jax import lax
from jax.experimental import pallas as pl
from jax.experimental.pallas import tpu as pltpu
from jax.experimental.pallas import tpu_sc as plsc

K = 20
M = 20
GROUP = 4
IN_CX = 32
BLK1 = 128
BLK3 = 128


def _knn_body(q_ref, k_ref, idx_ref, xnx_ref, xny_ref, xnz_ref, mn_ref, mx_ref):
    b = pl.program_id(0)
    n_total = k_ref.shape[1]
    q = q_ref[0]
    kk = k_ref[0]
    inner = -2.0 * lax.dot_general(q, kk, (((1,), (1,)), ((), ())),
                                   preferred_element_type=jnp.float32)
    sqq = jnp.sum(q * q, axis=1, keepdims=True)
    sqk = jnp.sum(kk * kk, axis=1)[None, :]
    vals = -sqq - inner - sqk
    iota = lax.broadcasted_iota(jnp.int32, vals.shape, 1)
    idx_cols = []
    xs_cols, ys_cols, zs_cols = [], [], []
    for _ in range(K):
        m = jnp.max(vals, axis=1, keepdims=True)
        sel = vals == m
        it = jnp.min(jnp.where(sel, iota, jnp.int32(2 ** 30)), axis=1,
                     keepdims=True)
        oh = (iota == it).astype(jnp.float32)
        xsel = lax.dot_general(oh, kk, (((1,), (0,)), ((), ())),
                               precision=lax.Precision.HIGHEST,
                               preferred_element_type=jnp.float32)
        vals = jnp.where(iota == it, -jnp.inf, vals)
        idx_cols.append(it)
        xs_cols.append(xsel[:, 0:1])
        ys_cols.append(xsel[:, 1:2])
        zs_cols.append(xsel[:, 2:3])
    idx = jnp.concatenate(idx_cols, axis=1)
    xnx = jnp.concatenate(xs_cols, axis=1)
    xny = jnp.concatenate(ys_cols, axis=1)
    xnz = jnp.concatenate(zs_cols, axis=1)
    idx_ref[0] = idx + b * n_total
    xnx_ref[0] = xnx
    xny_ref[0] = xny
    xnz_ref[0] = xnz
    relx = xnx - xnx[:, 0:1]
    rely = xny - xny[:, 0:1]
    relz = xnz - xnz[:, 0:1]
    dis = jnp.sqrt(relx * relx + rely * rely + relz * relz)
    mn = jnp.minimum(
        jnp.minimum(jnp.minimum(jnp.min(xnx[:, 0:1]), jnp.min(xny[:, 0:1])),
                    jnp.minimum(jnp.min(xnz[:, 0:1]), jnp.min(relx))),
        jnp.minimum(jnp.minimum(jnp.min(rely), jnp.min(relz)), jnp.min(dis)))
    mx = jnp.maximum(
        jnp.maximum(jnp.maximum(jnp.max(xnx[:, 0:1]), jnp.max(xny[:, 0:1])),
                    jnp.maximum(jnp.max(xnz[:, 0:1]), jnp.max(relx))),
        jnp.maximum(jnp.maximum(jnp.max(rely), jnp.max(relz)), jnp.max(dis)))
    mn_ref[0, 0] = jnp.full((1, 128), mn, jnp.float32)
    mx_ref[0, 0] = jnp.full((1, 128), mx, jnp.float32)


def _sc_gather(table, idx2d, n_rows):
    info = plsc.get_sparse_core_info()
    nw = info.num_cores * info.num_subcores
    nc = info.num_cores
    t_rows = n_rows
    per_w = t_rows // nw
    chunk = 512
    outer = per_w // chunk
    d = table.shape[1]

    @functools.partial(
        pl.kernel,
        mesh=plsc.VectorSubcoreMesh(core_axis_name="c", subcore_axis_name="s"),
        out_type=jax.ShapeDtypeStruct((t_rows, d), jnp.float32),
        scratch_types=[
            pltpu.VMEM((chunk // 128, 128), jnp.int32),
            pltpu.VMEM((chunk, d), jnp.float32),
            pltpu.SemaphoreType.DMA,
        ],
    )
    def gather_k(tab_hbm, idx_hbm, out_hbm, idx_v, rows_v, sem):
        wid = lax.axis_index("s") * nc + lax.axis_index("c")
        row_base = wid * per_w
        irow_base = wid * (per_w // 128)

        def body(j, carry):
            st = row_base + j * chunk
            pltpu.sync_copy(
                idx_hbm.at[pl.ds(irow_base + j * (chunk // 128), chunk // 128)],
                idx_v)
            cps = []
            for ii in range(chunk // 128):
                cps.append(pltpu.async_copy(
                    tab_hbm.at[idx_v.at[ii]],
                    rows_v.at[pl.ds(ii * 128, 128)], sem))
            for cp in cps:
                cp.wait()
            pltpu.sync_copy(rows_v, out_hbm.at[pl.ds(st, chunk)])
            return carry

        lax.fori_loop(0, outer, body, 0)

    return gather_k(table, idx2d)


def _dense_body(g_ref, xnx_ref, xny_ref, xnz_ref, f_ref, s_ref,
                bmap_ref, q_ref, pad_ref, mlpw_ref, mlpb_ref, w2b_ref,
                w2d_ref, c1b_ref, mow_ref, res_ref):
    blk = xnx_ref.shape[1]
    nfeat = f_ref.shape[1]
    fg = g_ref[0][:, :, :nfeat]
    f0 = fg[:, 0, :]
    xnx, xny, xnz = xnx_ref[0], xny_ref[0], xnz_ref[0]
    relx = xnx - xnx[:, 0:1]
    rely = xny - xny[:, 0:1]
    relz = xnz - xnz[:, 0:1]
    dis = jnp.sqrt(relx * relx + rely * rely + relz * relz)
    sval = s_ref[0, 0]
    two_pi = 2.0 * math.pi

    def enc(comp):
        return two_pi * (comp - sval)

    comps = [enc(jnp.broadcast_to(xnx[:, 0:1], (blk, K))),
             enc(jnp.broadcast_to(xny[:, 0:1], (blk, K))),
             enc(jnp.broadcast_to(xnz[:, 0:1], (blk, K))),
             enc(relx), enc(rely), enc(relz), enc(dis)]
    x7 = jnp.concatenate([cmp_[:, :, None] for cmp_ in comps],
                         axis=2).reshape(blk * K, 7)
    xe2 = lax.dot_general(x7, bmap_ref[...], (((1,), (0,)), ((), ())),
                          preferred_element_type=jnp.float32)
    e_sc = jnp.concatenate([jnp.sin(xe2), jnp.cos(xe2)], axis=1)
    xf2 = (lax.dot_general(e_sc, mlpw_ref[...], (((1,), (1,)), ((), ())),
                           preferred_element_type=jnp.float32)
           + mlpb_ref[0][None, :])
    fg2 = fg.reshape(blk * K, fg.shape[2])
    f2 = jnp.concatenate([fg2, xf2], axis=1)
    xf3 = xf2.reshape(blk, K, IN_CX)
    f0full = jnp.concatenate([f0, xf3[:, 0, :]], axis=1)
    g2 = lax.dot_general(f2, w2b_ref[...], (((1,), (1,)), ((), ())),
                         preferred_element_type=jnp.float32)
    h = lax.dot_general(f0full, w2d_ref[...], (((1,), (1,)), ((), ())),
                        preferred_element_type=jnp.float32)
    g3 = g2.reshape(blk, K, g2.shape[1]) + h[:, None, :]
    xr3 = jnp.concatenate([relx[:, :, None], rely[:, :, None],
                           relz[:, :, None]], axis=2).reshape(blk * K, 3)
    perm = lax.dot_general(xr3, q_ref[...], (((1,), (0,)), ((), ())),
                           preferred_element_type=jnp.float32)
    perm = perm.reshape(blk, K, M) + pad_ref[...][None]
    pmax = jnp.max(perm, axis=1, keepdims=True)
    pe = jnp.exp(perm - pmax)
    p = pe / jnp.sum(pe, axis=1, keepdims=True)
    p = jnp.where(p > 0.1, p, 0.0)
    pn = p / (jnp.sum(p, axis=1, keepdims=True) + 1e-6)
    g3t = g3.astype(jnp.bfloat16).astype(jnp.float32)
    pnt = pn.astype(jnp.bfloat16).astype(jnp.float32)
    acc = jnp.full((blk, g2.shape[1]), -jnp.inf, jnp.float32)
    for m in range(M):
        acc = jnp.maximum(acc, jnp.sum(g3t * pnt[:, :, m:m + 1], axis=1))
    out = acc + c1b_ref[0][None, :]
    out = 0.5 * out * (1.0 + lax.erf(out / math.sqrt(2.0)))
    fb = f_ref[0]
    r = lax.dot_general(mow_ref[...], fb, (((1,), (0,)), ((), ())),
                        preferred_element_type=jnp.float32)
    res_ref[0] = out.T + r


def _bn_body(res_ref, g_ref, b_ref, out_ref):
    res = res_ref[...]
    mean = jnp.mean(res, axis=(0, 2), keepdims=True)
    var = jnp.mean((res - mean) * (res - mean), axis=(0, 2), keepdims=True)
    gam = g_ref[0][None, :, None]
    bet = b_ref[0][None, :, None]
    out_ref[...] = (res - mean) / jnp.sqrt(var + 1e-5) * gam + bet


def kernel(x, feature, A, Bmap, kernels, one_padding, mlp_w, mlp_b, conv1_w,
           conv1_b, bn_gamma, bn_beta, mlp_out_w):
    b, c, n = feature.shape
    nb1 = n // BLK1
    xp = jnp.transpose(x, (0, 2, 1))

    idxg, xnx, xny, xnz, mns, mxs = pl.pallas_call(
        _knn_body,
        grid=(b, nb1),
        in_specs=[
            pl.BlockSpec((1, BLK1, 3), lambda i, j: (i, j, 0)),
            pl.BlockSpec((1, n, 3), lambda i, j: (i, 0, 0)),
        ],
        out_specs=[
            pl.BlockSpec((1, BLK1, K), lambda i, j: (i, j, 0)),
            pl.BlockSpec((1, BLK1, K), lambda i, j: (i, j, 0)),
            pl.BlockSpec((1, BLK1, K), lambda i, j: (i, j, 0)),
            pl.BlockSpec((1, BLK1, K), lambda i, j: (i, j, 0)),
            pl.BlockSpec((1, 1, 1, 128), lambda i, j: (i, j, 0, 0)),
            pl.BlockSpec((1, 1, 1, 128), lambda i, j: (i, j, 0, 0)),
        ],
        out_shape=[
            jax.ShapeDtypeStruct((b, n, K), jnp.int32),
            jax.ShapeDtypeStruct((b, n, K), jnp.float32),
            jax.ShapeDtypeStruct((b, n, K), jnp.float32),
            jax.ShapeDtypeStruct((b, n, K), jnp.float32),
            jax.ShapeDtypeStruct((b, nb1, 1, 128), jnp.float32),
            jax.ShapeDtypeStruct((b, nb1, 1, 128), jnp.float32),
        ],
    )(xp, xp)

    mn = jnp.min(mns)
    mx = jnp.max(mxs)
    s = (mn / (mx - mn)).reshape(1, 1)

    table = jnp.transpose(feature, (0, 2, 1)).reshape(b * n, c)
    table = jnp.concatenate([table, jnp.zeros_like(table)], axis=1)
    idx2d = idxg.reshape(b * n * K // 128, 128)
    gathered = _sc_gather(table, idx2d, b * n * K)
    g4 = gathered.reshape(b, n, K, 2 * c)

    nf = (c + IN_CX) * 2
    gsz = nf // GROUP
    inv = np.array([(j % gsz) * GROUP + j // gsz for j in range(nf)])
    w2 = conv1_w[:, inv]
    w2a, w2b = w2[:, :nf // 2], w2[:, nf // 2:]
    w2d = w2a - w2b
    q_mat = ((A + A.T) / 2.0) @ kernels

    nb3 = n // BLK3
    res = pl.pallas_call(
        _dense_body,
        grid=(b, nb3),
        in_specs=[
            pl.BlockSpec((1, BLK3, K, 2 * c), lambda i, j: (i, j, 0, 0)),
            pl.BlockSpec((1, BLK3, K), lambda i, j: (i, j, 0)),
            pl.BlockSpec((1, BLK3, K), lambda i, j: (i, j, 0)),
            pl.BlockSpec((1, BLK3, K), lambda i, j: (i, j, 0)),
            pl.BlockSpec((1, c, BLK3), lambda i, j: (i, 0, j)),
            pl.BlockSpec((1, 1), lambda i, j: (0, 0)),
            pl.BlockSpec(Bmap.shape, lambda i, j: (0, 0)),
            pl.BlockSpec((3, M), lambda i, j: (0, 0)),
            pl.BlockSpec((K, M), lambda i, j: (0, 0)),
            pl.BlockSpec(mlp_w.shape, lambda i, j: (0, 0)),
            pl.BlockSpec((1, IN_CX), lambda i, j: (0, 0)),
            pl.BlockSpec((c, nf // 2), lambda i, j: (0, 0)),
            pl.BlockSpec((c, nf // 2), lambda i, j: (0, 0)),
            pl.BlockSpec((1, c), lambda i, j: (0, 0)),
            pl.BlockSpec((c, c), lambda i, j: (0, 0)),
        ],
        out_specs=pl.BlockSpec((1, c, BLK3), lambda i, j: (i, 0, j)),
        out_shape=jax.ShapeDtypeStruct((b, c, n), jnp.float32),
    )(g4, xnx, xny, xnz, feature, s, Bmap, q_mat, one_padding, mlp_w,
      mlp_b.reshape(1, IN_CX), w2b, w2d, conv1_b.reshape(1, c), mlp_out_w)

    out = pl.pallas_call(
        _bn_body,
        grid=(1,),
        in_specs=[
            pl.BlockSpec((b, c, n), lambda i: (0, 0, 0)),
            pl.BlockSpec((1, c), lambda i: (0, 0)),
            pl.BlockSpec((1, c), lambda i: (0, 0)),
        ],
        out_specs=pl.BlockSpec((b, c, n), lambda i: (0, 0, 0)),
        out_shape=jax.ShapeDtypeStruct((b, c, n), jnp.float32),
    )(res, bn_gamma.reshape(1, c), bn_beta.reshape(1, c))
    return out
